# baseline (speedup 1.0000x reference)
import functools

import jax
import jax.numpy as jnp
from jax import lax
from jax.experimental import pallas as pl
from jax.experimental.pallas import tpu as pltpu
from jax.experimental.pallas import tpu_sc as plsc

NC = 2
NS = 16
NW = NC * NS
L = 16


def _make_sc_kernel(B, S, D, P):
    BPW = B // NW
    DT = D // 8
    KK = D // L
    CHUNK = DT * 8 * BPW
    assert B % NW == 0 and D % L == 0 and S % 2 == 0 and BPW % L == 0

    mesh = plsc.VectorSubcoreMesh(core_axis_name="c", subcore_axis_name="s")

    @functools.partial(
        pl.kernel,
        out_type=jax.ShapeDtypeStruct((S, DT, NW, 8, BPW), jnp.float32),
        mesh=mesh,
        compiler_params=pltpu.CompilerParams(use_tc_tiling_on_sc=False,
                                             needs_layout_passes=False),
        scratch_types=[
            pltpu.VMEM((S, BPW), jnp.int32),
            pltpu.VMEM((P * D,), jnp.float32),
            [pltpu.VMEM((BPW, D), jnp.float32)] * 4,
            [pltpu.VMEM((D, BPW + 1), jnp.float32)] * 4,
            pltpu.VMEM((D,), jnp.int32),
            [pltpu.SemaphoreType.DMA] * 4,
            [pltpu.SemaphoreType.DMA] * 4,
        ],
    )
    def sc_kernel(xT_hbm, posf_hbm, tok_hbm, out_hbm,
                  idx_v, pos_v, ins, outs, cbase, gsems, osems):
        wid = lax.axis_index("s") * NC + lax.axis_index("c")
        b0 = wid * BPW

        pltpu.sync_copy(xT_hbm.at[:, pl.ds(b0, BPW)], idx_v)
        pltpu.sync_copy(posf_hbm, pos_v)

        lanes = lax.iota(jnp.int32, L)
        for k in range(KK):
            cbase[pl.ds(k * L, L)] = lanes + k * L

        def start_gather(s, inbuf, gsem):
            pltpu.async_copy(tok_hbm.at[idx_v.at[s]], inbuf, gsem)

        def gather_wait(inbuf, gsem):
            pltpu.make_async_copy(tok_hbm.at[idx_v.at[0]], inbuf, gsem).wait()

        def store_wait(outbuf, osem):
            for dt in range(DT):
                pltpu.make_async_copy(
                    outbuf.at[pl.ds(dt * 8, 8), pl.ds(0, BPW)],
                    out_hbm.at[0, dt, 0], osem).wait()

        def compute(s, inbuf, outbuf):
            svec = jnp.full((L,), s, dtype=jnp.int32)
            pbase = (s + 1) * D
            prow = [plsc.load_gather(pos_v, [pbase + k * L + lanes])
                    for k in range(KK)]
            colb = [cbase[pl.ds(k * L, L)] for k in range(KK)]
            RB = 4
            for rb in range(0, BPW, RB):
                rows = range(rb, rb + RB)
                xspl = [plsc.load_gather(
                    idx_v, [svec, jnp.full((L,), r, dtype=jnp.int32)])
                    for r in rows]
                ms = [jnp.where(xv > 0, jnp.float32(1.0), jnp.float32(0.0))
                      for xv in xspl]
                tvs = [inbuf[r, pl.ds(k * L, L)]
                       for r in rows for k in range(KK)]
                pms = [prow[k] * ms[i]
                       for i in range(RB) for k in range(KK)]
                vals = [tv + pm for tv, pm in zip(tvs, pms)]
                i = 0
                for r in rows:
                    rsp = jnp.full((L,), r, dtype=jnp.int32)
                    for k in range(KK):
                        plsc.store_scatter(outbuf, [colb[k], rsp], vals[i])
                        i += 1

        def start_store(s, outbuf, osem):
            for dt in range(DT):
                pltpu.async_copy(
                    outbuf.at[pl.ds(dt * 8, 8), pl.ds(0, BPW)],
                    out_hbm.at[s, dt, wid],
                    osem)

        NB = 4
        for j in range(NB):
            start_gather(j, ins[j], gsems[j])

        def body(it, carry):
            s0 = NB * it
            for j in range(NB):
                @pl.when(it > 0)
                def _(j=j):
                    store_wait(outs[j], osems[j])
                gather_wait(ins[j], gsems[j])
                compute(s0 + j, ins[j], outs[j])
                start_store(s0 + j, outs[j], osems[j])

                @pl.when(it < S // NB - 1)
                def _(j=j):
                    start_gather(s0 + j + NB, ins[j], gsems[j])
            return carry

        lax.fori_loop(0, S // NB, body, 0)
        for j in range(NB):
            store_wait(outs[j], osems[j])

    return sc_kernel


@jax.jit
def kernel(x, tok_table, pos_table):
    B, S = x.shape
    V, D = tok_table.shape
    P = pos_table.shape[0]
    xT = jnp.transpose(x)
    posf = pos_table.reshape(P * D)
    out5 = _make_sc_kernel(B, S, D, P)(xT, posf, tok_table)
    out = out5.transpose(2, 4, 0, 1, 3).reshape(B, S, D)
    return out


# speedup vs baseline: 1.0406x; 1.0406x over previous
import functools

import jax
import jax.numpy as jnp
from jax import lax
from jax.experimental import pallas as pl
from jax.experimental.pallas import tpu as pltpu
from jax.experimental.pallas import tpu_sc as plsc

NC = 2
NS = 16
NW = NC * NS
L = 16


def _make_sc_kernel(B, S, D, P):
    BPW = B // NW
    DT = D // 8
    KK = D // L
    CHUNK = DT * 8 * BPW
    assert B % NW == 0 and D % L == 0 and S % 2 == 0 and BPW % L == 0

    mesh = plsc.VectorSubcoreMesh(core_axis_name="c", subcore_axis_name="s")

    @functools.partial(
        pl.kernel,
        out_type=jax.ShapeDtypeStruct((S, DT, NW, 8, BPW), jnp.float32),
        mesh=mesh,
        compiler_params=pltpu.CompilerParams(use_tc_tiling_on_sc=False,
                                             needs_layout_passes=False),
        scratch_types=[
            pltpu.VMEM((S, BPW), jnp.int32),
            pltpu.VMEM((P * D,), jnp.float32),
            [pltpu.VMEM((BPW, D), jnp.float32)] * 2,
            [pltpu.VMEM((D, BPW + 1), jnp.float32)] * 2,
            pltpu.VMEM((D,), jnp.int32),
            [pltpu.SemaphoreType.DMA] * 2,
            [pltpu.SemaphoreType.DMA] * 2,
        ],
    )
    def sc_kernel(xT_hbm, posf_hbm, tok_hbm, out_hbm,
                  idx_v, pos_v, ins, outs, cbase, gsems, osems):
        wid = lax.axis_index("s") * NC + lax.axis_index("c")
        b0 = wid * BPW

        pltpu.sync_copy(xT_hbm.at[:, pl.ds(b0, BPW)], idx_v)
        pltpu.sync_copy(posf_hbm, pos_v)

        lanes = lax.iota(jnp.int32, L)
        for k in range(KK):
            cbase[pl.ds(k * L, L)] = lanes + k * L

        def start_gather(s, inbuf, gsem):
            pltpu.async_copy(tok_hbm.at[idx_v.at[s]], inbuf, gsem)

        def gather_wait(inbuf, gsem):
            pltpu.make_async_copy(tok_hbm.at[idx_v.at[0]], inbuf, gsem).wait()

        def store_wait(outbuf, osem):
            for dt in range(DT):
                pltpu.make_async_copy(
                    outbuf.at[pl.ds(dt * 8, 8), pl.ds(0, BPW)],
                    out_hbm.at[0, dt, 0], osem).wait()

        def compute(s, inbuf, outbuf):
            svec = jnp.full((L,), s, dtype=jnp.int32)
            pbase = (s + 1) * D
            prow = [plsc.load_gather(pos_v, [pbase + k * L + lanes])
                    for k in range(KK)]
            colb = [cbase[pl.ds(k * L, L)] for k in range(KK)]
            RB = 8
            for rb in range(0, BPW, RB):
                rows = range(rb, rb + RB)
                xspl = [plsc.load_gather(
                    idx_v, [svec, jnp.full((L,), r, dtype=jnp.int32)])
                    for r in rows]
                ms = [jnp.where(xv > 0, jnp.float32(1.0), jnp.float32(0.0))
                      for xv in xspl]
                tvs = [inbuf[r, pl.ds(k * L, L)]
                       for r in rows for k in range(KK)]
                pms = [prow[k] * ms[i]
                       for i in range(RB) for k in range(KK)]
                vals = [tv + pm for tv, pm in zip(tvs, pms)]
                i = 0
                for r in rows:
                    rsp = jnp.full((L,), r, dtype=jnp.int32)
                    for k in range(KK):
                        plsc.store_scatter(outbuf, [colb[k], rsp], vals[i])
                        i += 1

        def start_store(s, outbuf, osem):
            for dt in range(DT):
                pltpu.async_copy(
                    outbuf.at[pl.ds(dt * 8, 8), pl.ds(0, BPW)],
                    out_hbm.at[s, dt, wid], osem)

        NB = 2
        for j in range(NB):
            start_gather(j, ins[j], gsems[j])

        def body(it, carry):
            s0 = NB * it
            for j in range(NB):
                @pl.when(it > 0)
                def _(j=j):
                    store_wait(outs[j], osems[j])
                gather_wait(ins[j], gsems[j])
                compute(s0 + j, ins[j], outs[j])
                start_store(s0 + j, outs[j], osems[j])

                @pl.when(it < S // NB - 1)
                def _(j=j):
                    start_gather(s0 + j + NB, ins[j], gsems[j])
            return carry

        lax.fori_loop(0, S // NB, body, 0)
        for j in range(NB):
            store_wait(outs[j], osems[j])

    return sc_kernel


@jax.jit
def kernel(x, tok_table, pos_table):
    B, S = x.shape
    V, D = tok_table.shape
    P = pos_table.shape[0]
    xT = jnp.transpose(x)
    posf = pos_table.reshape(P * D)
    out5 = _make_sc_kernel(B, S, D, P)(xT, posf, tok_table)
    out = out5.transpose(2, 4, 0, 1, 3).reshape(B, S, D)
    return out


# speedup vs baseline: 1.0644x; 1.0229x over previous
import functools

import jax
import jax.numpy as jnp
from jax import lax
from jax.experimental import pallas as pl
from jax.experimental.pallas import tpu as pltpu
from jax.experimental.pallas import tpu_sc as plsc

NC = 2
NS = 16
NW = NC * NS
L = 16


def _make_sc_kernel(B, S, D, P):
    BPW = B // NW
    DT = D // 8
    KK = D // L
    CHUNK = DT * 8 * BPW
    assert B % NW == 0 and D % L == 0 and S % 2 == 0 and BPW % L == 0

    mesh = plsc.VectorSubcoreMesh(core_axis_name="c", subcore_axis_name="s")

    @functools.partial(
        pl.kernel,
        out_type=jax.ShapeDtypeStruct((S, DT, NW, 8, BPW), jnp.float32),
        mesh=mesh,
        compiler_params=pltpu.CompilerParams(use_tc_tiling_on_sc=False,
                                             needs_layout_passes=False),
        scratch_types=[
            pltpu.VMEM((S, BPW), jnp.int32),
            pltpu.VMEM((P * D,), jnp.float32),
            [pltpu.VMEM((BPW, D), jnp.float32)] * 2,
            [pltpu.VMEM((DT, 8, BPW + 1), jnp.float32)] * 2,
            pltpu.VMEM((D + L,), jnp.int32),
            [pltpu.SemaphoreType.DMA] * 2,
            [pltpu.SemaphoreType.DMA] * 2,
        ],
    )
    def sc_kernel(xT_hbm, posf_hbm, tok_hbm, out_hbm,
                  idx_v, pos_v, ins, outs, cbase, gsems, osems):
        wid = lax.axis_index("s") * NC + lax.axis_index("c")
        b0 = wid * BPW

        pltpu.sync_copy(xT_hbm.at[:, pl.ds(b0, BPW)], idx_v)
        pltpu.sync_copy(posf_hbm, pos_v)

        lanes = lax.iota(jnp.int32, L)
        for k in range(KK):
            cbase[pl.ds(k * L, L)] = lax.shift_right_logical(
                lanes + k * L, 3)
        cbase[pl.ds(D, L)] = lax.bitwise_and(lanes, 7)

        def start_gather(s, inbuf, gsem):
            pltpu.async_copy(tok_hbm.at[idx_v.at[s]], inbuf, gsem)

        def gather_wait(inbuf, gsem):
            pltpu.make_async_copy(tok_hbm.at[idx_v.at[0]], inbuf, gsem).wait()

        def store_wait(outbuf, osem):
            pltpu.make_async_copy(
                outbuf.at[:, :, pl.ds(0, BPW)],
                out_hbm.at[0, :, 0], osem).wait()

        def compute(s, inbuf, outbuf):
            svec = jnp.full((L,), s, dtype=jnp.int32)
            pbase = (s + 1) * D
            prow = [plsc.load_gather(pos_v, [pbase + k * L + lanes])
                    for k in range(KK)]
            colb = [cbase[pl.ds(k * L, L)] for k in range(KK)]
            dsub = cbase[pl.ds(D, L)]
            RB = 8
            for rb in range(0, BPW, RB):
                rows = range(rb, rb + RB)
                xspl = [plsc.load_gather(
                    idx_v, [svec, jnp.full((L,), r, dtype=jnp.int32)])
                    for r in rows]
                ms = [jnp.where(xv > 0, jnp.float32(1.0), jnp.float32(0.0))
                      for xv in xspl]
                tvs = [inbuf[r, pl.ds(k * L, L)]
                       for r in rows for k in range(KK)]
                pms = [prow[k] * ms[i]
                       for i in range(RB) for k in range(KK)]
                vals = [tv + pm for tv, pm in zip(tvs, pms)]
                i = 0
                for r in rows:
                    rsp = jnp.full((L,), r, dtype=jnp.int32)
                    for k in range(KK):
                        plsc.store_scatter(
                            outbuf, [colb[k], dsub, rsp], vals[i])
                        i += 1

        def start_store(s, outbuf, osem):
            pltpu.async_copy(
                outbuf.at[:, :, pl.ds(0, BPW)],
                out_hbm.at[s, :, wid], osem)

        NB = 2
        for j in range(NB):
            start_gather(j, ins[j], gsems[j])

        def body(it, carry):
            s0 = NB * it
            for j in range(NB):
                @pl.when(it > 0)
                def _(j=j):
                    store_wait(outs[j], osems[j])
                gather_wait(ins[j], gsems[j])
                compute(s0 + j, ins[j], outs[j])
                start_store(s0 + j, outs[j], osems[j])

                @pl.when(it < S // NB - 1)
                def _(j=j):
                    start_gather(s0 + j + NB, ins[j], gsems[j])
            return carry

        lax.fori_loop(0, S // NB, body, 0)
        for j in range(NB):
            store_wait(outs[j], osems[j])

    return sc_kernel


@jax.jit
def kernel(x, tok_table, pos_table):
    B, S = x.shape
    V, D = tok_table.shape
    P = pos_table.shape[0]
    xT = jnp.transpose(x)
    posf = pos_table.reshape(P * D)
    out5 = _make_sc_kernel(B, S, D, P)(xT, posf, tok_table)
    out = out5.transpose(2, 4, 0, 1, 3).reshape(B, S, D)
    return out


# speedup vs baseline: 1.1839x; 1.1122x over previous
import functools

import jax
import jax.numpy as jnp
from jax import lax
from jax.experimental import pallas as pl
from jax.experimental.pallas import tpu as pltpu
from jax.experimental.pallas import tpu_sc as plsc

NC = 2
NS = 16
NW = NC * NS
L = 16


def _make_sc_kernel(B, S, D, P):
    BPW = B // NW
    RPB = 16
    DT = D // 8
    KK = D // L
    CHUNK = DT * 8 * BPW
    assert B % NW == 0 and D % L == 0 and S % 2 == 0 and BPW % L == 0

    mesh = plsc.VectorSubcoreMesh(core_axis_name="c", subcore_axis_name="s")

    @functools.partial(
        pl.kernel,
        out_type=jax.ShapeDtypeStruct((S, DT, NW, 8, BPW), jnp.float32),
        mesh=mesh,
        compiler_params=pltpu.CompilerParams(use_tc_tiling_on_sc=False,
                                             needs_layout_passes=False),
        scratch_types=[
            pltpu.VMEM((S, BPW), jnp.int32),
            pltpu.VMEM((P * D,), jnp.float32),
            [pltpu.VMEM((BPW, D), jnp.float32)] * 4,
            [pltpu.VMEM((DT, 8, BPW + 1), jnp.float32)] * 4,
            pltpu.VMEM((D + L,), jnp.int32),
            [pltpu.SemaphoreType.DMA] * 4,
            [pltpu.SemaphoreType.DMA] * 4,
        ],
    )
    def sc_kernel(xT_hbm, posf_hbm, tok_hbm, out_hbm,
                  idx_v, pos_v, ins, outs, cbase, gsems, osems):
        wid = lax.axis_index("s") * NC + lax.axis_index("c")
        b0 = wid * BPW

        pltpu.sync_copy(xT_hbm.at[:, pl.ds(b0, BPW)], idx_v)
        pltpu.sync_copy(posf_hbm, pos_v)

        lanes = lax.iota(jnp.int32, L)
        for k in range(KK):
            cbase[pl.ds(k * L, L)] = lax.shift_right_logical(
                lanes + k * L, 3)
        cbase[pl.ds(D, L)] = lax.bitwise_and(lanes, 7)

        def start_gather(s, inbuf, gsem):
            pltpu.async_copy(tok_hbm.at[idx_v.at[s]], inbuf, gsem)

        def gather_wait(inbuf, gsem):
            pltpu.make_async_copy(tok_hbm.at[idx_v.at[0]], inbuf, gsem).wait()

        def store_wait(outbuf, osem):
            pltpu.make_async_copy(
                outbuf.at[:, :, pl.ds(0, BPW)],
                out_hbm.at[0, :, 0], osem).wait()

        def compute(s, inbuf, outbuf):
            svec = jnp.full((L,), s, dtype=jnp.int32)
            pbase = (s + 1) * D
            prow = [plsc.load_gather(pos_v, [pbase + k * L + lanes])
                    for k in range(KK)]
            colb = [cbase[pl.ds(k * L, L)] for k in range(KK)]
            dsub = cbase[pl.ds(D, L)]
            RB = 4

            def rowblock(blk, carry):
                r0 = blk * RPB
                for rb in range(0, RPB, RB):
                    rows = [r0 + rb + q for q in range(RB)]
                    xspl = [plsc.load_gather(
                        idx_v, [svec, jnp.full((L,), r, dtype=jnp.int32)])
                        for r in rows]
                    ms = [jnp.where(xv > 0, jnp.float32(1.0),
                                    jnp.float32(0.0)) for xv in xspl]
                    tvs = [inbuf[r, pl.ds(k * L, L)]
                           for r in rows for k in range(KK)]
                    pms = [prow[k] * ms[i]
                           for i in range(RB) for k in range(KK)]
                    vals = [tv + pm for tv, pm in zip(tvs, pms)]
                    i = 0
                    for r in rows:
                        rsp = jnp.full((L,), r, dtype=jnp.int32)
                        for k in range(KK):
                            plsc.store_scatter(
                                outbuf, [colb[k], dsub, rsp], vals[i])
                            i += 1
                return carry

            lax.fori_loop(0, BPW // RPB, rowblock, 0)

        def start_store(s, outbuf, osem):
            pltpu.async_copy(
                outbuf.at[:, :, pl.ds(0, BPW)],
                out_hbm.at[s, :, wid], osem)

        NB = 4
        for j in range(NB):
            start_gather(j, ins[j], gsems[j])

        def body(it, carry):
            s0 = NB * it
            for j in range(NB):
                @pl.when(it > 0)
                def _(j=j):
                    store_wait(outs[j], osems[j])
                gather_wait(ins[j], gsems[j])
                compute(s0 + j, ins[j], outs[j])
                start_store(s0 + j, outs[j], osems[j])

                @pl.when(it < S // NB - 1)
                def _(j=j):
                    start_gather(s0 + j + NB, ins[j], gsems[j])
            return carry

        lax.fori_loop(0, S // NB, body, 0)
        for j in range(NB):
            store_wait(outs[j], osems[j])

    return sc_kernel


@jax.jit
def kernel(x, tok_table, pos_table):
    B, S = x.shape
    V, D = tok_table.shape
    P = pos_table.shape[0]
    xT = jnp.transpose(x)
    posf = pos_table.reshape(P * D)
    out5 = _make_sc_kernel(B, S, D, P)(xT, posf, tok_table)
    out = out5.transpose(2, 4, 0, 1, 3).reshape(B, S, D)
    return out
